# Initial kernel scaffold; baseline (speedup 1.0000x reference)
import jax
import jax.numpy as jnp
from jax.experimental import pallas as pl


def kernel(probs):
    raise NotImplementedError("write your pallas kernel here")


# speedup vs baseline: 2.4430x; 2.4430x over previous
import jax
import jax.numpy as jnp
from jax.experimental import pallas as pl
from jax.experimental.pallas import tpu as pltpu

R, C = 64, 32768
BLK = 4096
NB = C // BLK


def _body(x_ref, o_ref, max_s, idx_s):
    p = pl.program_id(0)
    j = pl.program_id(1)

    @pl.when(p == 0)
    def _phase0():
        @pl.when(j == 0)
        def _init():
            max_s[...] = jnp.full((R, 1), -jnp.inf, jnp.float32)
            idx_s[...] = jnp.zeros((R, 1), jnp.int32)

        x = x_ref[...]
        bm = jnp.max(x, axis=1, keepdims=True)
        bi = jnp.argmax(x, axis=1).astype(jnp.int32)[:, None] + j * BLK
        upd = bm > max_s[...]
        idx_s[...] = jnp.where(upd, bi, idx_s[...])
        max_s[...] = jnp.where(upd, bm, max_s[...])

    @pl.when(p == 1)
    def _phase1():
        cols = jax.lax.broadcasted_iota(jnp.int32, (R, BLK), 1) + j * BLK
        o_ref[...] = jnp.where(cols == idx_s[...], 1.0, 0.0).astype(jnp.float32)


def kernel(probs):
    return pl.pallas_call(
        _body,
        grid=(2, NB),
        in_specs=[pl.BlockSpec((R, BLK), lambda p, j: (0, j * (1 - p)))],
        out_specs=pl.BlockSpec((R, BLK), lambda p, j: (0, j * p)),
        out_shape=jax.ShapeDtypeStruct((R, C), jnp.float32),
        scratch_shapes=[
            pltpu.VMEM((R, 1), jnp.float32),
            pltpu.VMEM((R, 1), jnp.int32),
        ],
    )(probs)
